# baseline (speedup 1.0000x reference)
import functools

import jax
import jax.numpy as jnp
from jax import lax
from jax.experimental import pallas as pl
from jax.experimental.pallas import tpu as pltpu
from jax.experimental.pallas import tpu_sc as plsc

N_USERS = 25000
N_ITEMS = 75000
N_NODES = N_USERS + N_ITEMS
EMBED_DIM = 32
N_EDGES = 1_600_000
BATCH = 4096

NUM_SC = 2
TILES = 16
HALF = N_NODES // NUM_SC
ACC_ROWS = 50176
TRASH = HALF
TILE_EDGES = N_EDGES // TILES
EC = 400
SUB = 80
NSUB = EC // SUB
NGRP = SUB // 16
NCHUNK = TILE_EDGES // EC
ZROWS = 112
ZITER = (ACC_ROWS // TILES) // ZROWS
WROWS = 80
WCHUNKS = HALF // WROWS
WITER = (WCHUNKS + TILES - 1) // TILES
BPT = BATCH // (NUM_SC * TILES)

_mesh = plsc.VectorSubcoreMesh(core_axis_name="c", subcore_axis_name="s")


@functools.partial(
    pl.kernel,
    out_type=jax.ShapeDtypeStruct((N_NODES, EMBED_DIM), jnp.float32),
    mesh=_mesh,
    scratch_types=[
        pltpu.VMEM((3, EC), jnp.int32),
        pltpu.VMEM((3, EC), jnp.int32),
        pltpu.VMEM((3, EC), jnp.float32),
        pltpu.VMEM((2, NSUB, SUB), jnp.int32),
        pltpu.VMEM((2, EC, EMBED_DIM), jnp.float32),
        pltpu.VMEM_SHARED((ACC_ROWS, EMBED_DIM), jnp.float32),
        pltpu.SemaphoreType.DMA,
        pltpu.SemaphoreType.DMA,
        pltpu.SemaphoreType.DMA,
    ],
    compiler_params=pltpu.CompilerParams(use_tc_tiling_on_sc=False),
)
def _spmm(emb, rowa, cola, wa, out, rowb, colb, wb, idxb, rows, acc,
          sem_lin, sem_g, sem_s):
    c = lax.axis_index("c")
    s = lax.axis_index("s")
    lo = c * HALF

    def _zrow(i, carry):
        z = jnp.zeros((16,), jnp.float32)
        rows[0, i, pl.ds(0, 16)] = z
        rows[0, i, pl.ds(16, 16)] = z
        return carry
    lax.fori_loop(0, 392, _zrow, 0)

    zn = (ACC_ROWS // TILES) // 392
    zdescs = [
        pltpu.async_copy(
            rows.at[0, pl.ds(0, 392)],
            acc.at[pl.ds(s * (ACC_ROWS // TILES) + i * 392, 392)], sem_g)
        for i in range(zn)
    ]
    for zd in zdescs:
        zd.wait()
    plsc.subcore_barrier()

    base0 = s * TILE_EDGES

    def _issue_lin(ci):
        b = base0 + ci * EC
        slot = ci % 3
        pltpu.async_copy(rowa.at[pl.ds(b, EC)], rowb.at[slot], sem_lin)
        pltpu.async_copy(cola.at[pl.ds(b, EC)], colb.at[slot], sem_lin)
        pltpu.async_copy(wa.at[pl.ds(b, EC)], wb.at[slot], sem_lin)

    def _wait_lin():
        pltpu.make_async_copy(rowa.at[pl.ds(0, EC)], rowb.at[0], sem_lin).wait()
        pltpu.make_async_copy(cola.at[pl.ds(0, EC)], colb.at[0], sem_lin).wait()
        pltpu.make_async_copy(wa.at[pl.ds(0, EC)], wb.at[0], sem_lin).wait()

    def _issue_gathers(ci, buf):
        slot = ci % 3
        for k in range(NSUB):
            pltpu.async_copy(emb.at[colb.at[slot, pl.ds(k * SUB, SUB)]],
                             rows.at[buf, pl.ds(k * SUB, SUB)], sem_g)

    _issue_lin(0)
    _issue_lin(1)
    _wait_lin()
    _issue_gathers(0, 0)

    def _drain_scatters(buf):
        pltpu.make_async_copy(rows.at[buf], acc.at[pl.ds(0, EC)], sem_s).wait()

    def _chunk(ci, carry):
        cur = lax.rem(ci, 2)
        nxt = 1 - cur
        slot = lax.rem(ci, 3)

        @pl.when(ci >= 1)
        def _():
            _drain_scatters(nxt)

        @pl.when(ci + 1 < NCHUNK)
        def _():
            _wait_lin()
            _issue_gathers_dyn(ci + 1, nxt)

            @pl.when(ci + 2 < NCHUNK)
            def _():
                _issue_lin_dyn(ci + 2)

        pltpu.make_async_copy(emb.at[pl.ds(0, EC)], rows.at[cur], sem_g).wait()

        def _sub(k, carry2):
            def _grp(g, carry3):
                off = k * SUB + g * 16
                r = rowb[slot, pl.ds(off, 16)]
                w = wb[slot, pl.ds(off, 16)]
                inb = (r >= lo) & (r < lo + HALF)
                idx = jnp.where(inb, r - lo, TRASH)
                idxb[cur, k, pl.ds(g * 16, 16)] = idx
                for j in range(16):
                    e = off + j
                    wj = w.at[jnp.full((16,), j, jnp.int32)].get(
                        mode="promise_in_bounds")
                    rows[cur, e, pl.ds(0, 16)] = rows[cur, e, pl.ds(0, 16)] * wj
                    rows[cur, e, pl.ds(16, 16)] = (rows[cur, e, pl.ds(16, 16)]
                                                   * wj)
                return carry3
            lax.fori_loop(0, NGRP, _grp, 0)
            pltpu.async_copy(rows.at[cur, pl.ds(k * SUB, SUB)],
                             acc.at[idxb.at[cur, k]], sem_s, add=True)
            return carry2
        lax.fori_loop(0, NSUB, _sub, 0)
        return carry

    def _issue_lin_dyn(ci):
        b = base0 + ci * EC
        slot = lax.rem(ci, 3)
        pltpu.async_copy(rowa.at[pl.ds(b, EC)], rowb.at[slot], sem_lin)
        pltpu.async_copy(cola.at[pl.ds(b, EC)], colb.at[slot], sem_lin)
        pltpu.async_copy(wa.at[pl.ds(b, EC)], wb.at[slot], sem_lin)

    def _issue_gathers_dyn(ci, buf):
        slot = lax.rem(ci, 3)
        for k in range(NSUB):
            pltpu.async_copy(emb.at[colb.at[slot, pl.ds(k * SUB, SUB)]],
                             rows.at[buf, pl.ds(k * SUB, SUB)], sem_g)

    lax.fori_loop(0, NCHUNK, _chunk, 0)
    _drain_scatters(lax.rem(NCHUNK - 1, 2))
    plsc.subcore_barrier()

    def _wb(i, carry):
        j = s + i * TILES

        @pl.when(j < WCHUNKS)
        def _():
            src = j * WROWS
            pltpu.sync_copy(acc.at[pl.ds(src, WROWS)],
                            rows.at[0, pl.ds(0, WROWS)])
            pltpu.sync_copy(rows.at[0, pl.ds(0, WROWS)],
                            out.at[pl.ds(lo + src, WROWS)])
        return carry
    lax.fori_loop(0, WITER, _wb, 0)


@functools.partial(
    pl.kernel,
    out_type=(jax.ShapeDtypeStruct((BATCH, EMBED_DIM), jnp.float32),) * 3,
    mesh=_mesh,
    scratch_types=[
        pltpu.VMEM((BPT,), jnp.int32),
        pltpu.VMEM((BPT, EMBED_DIM), jnp.float32),
        pltpu.VMEM((BPT, EMBED_DIM), jnp.float32),
        pltpu.VMEM((BPT, EMBED_DIM), jnp.float32),
        pltpu.VMEM((BPT, EMBED_DIM), jnp.float32),
        pltpu.SemaphoreType.DMA,
    ],
    compiler_params=pltpu.CompilerParams(use_tc_tiling_on_sc=False),
)
def _final(t0, t1, t2, t3, usr, pos, neg, ou, op, on, idxb, b0, b1, b2, b3,
           sem):
    c = lax.axis_index("c")
    s = lax.axis_index("s")
    base = (s * NUM_SC + c) * BPT
    for ids, off, outref in ((usr, 0, ou), (pos, N_USERS, op),
                             (neg, N_USERS, on)):
        pltpu.sync_copy(ids.at[pl.ds(base, BPT)], idxb)
        if off:
            def _adj(g, carry):
                idxb[pl.ds(g * 16, 16)] = idxb[pl.ds(g * 16, 16)] + off
                return carry
            lax.fori_loop(0, BPT // 16, _adj, 0)
        descs = [pltpu.async_copy(t.at[idxb], bb, sem)
                 for t, bb in ((t0, b0), (t1, b1), (t2, b2), (t3, b3))]
        for d in descs:
            d.wait()

        def _mean(r, carry):
            for h in (0, 16):
                v = (b0[r, pl.ds(h, 16)] + b1[r, pl.ds(h, 16)]
                     + b2[r, pl.ds(h, 16)] + b3[r, pl.ds(h, 16)]) * 0.25
                b0[r, pl.ds(h, 16)] = v
            return carry
        lax.fori_loop(0, BPT, _mean, 0)
        pltpu.sync_copy(b0, outref.at[pl.ds(base, BPT)])


def kernel(users, pos_items, neg_items, edge_index, edge_weight, user_emb,
           item_emb):
    row = edge_index[0]
    col = edge_index[1]
    e0 = jnp.concatenate([user_emb, item_emb], axis=0)
    e1 = _spmm(e0, row, col, edge_weight)
    e2 = _spmm(e1, row, col, edge_weight)
    e3 = _spmm(e2, row, col, edge_weight)
    return _final(e0, e1, e2, e3, users, pos_items, neg_items)


# speedup vs baseline: 2.8039x; 2.8039x over previous
import functools

import jax
import jax.numpy as jnp
from jax import lax
from jax.experimental import pallas as pl
from jax.experimental.pallas import tpu as pltpu
from jax.experimental.pallas import tpu_sc as plsc

N_USERS = 25000
N_ITEMS = 75000
N_NODES = N_USERS + N_ITEMS
EMBED_DIM = 32
HDIM = EMBED_DIM // 2
N_EDGES = 1_600_000
BATCH = 4096

NUM_SC = 2
TILES = 16
ACC_ROWS = 100352
TILE_EDGES = N_EDGES // TILES
EC = 400
SUB = 80
NSUB = EC // SUB
NGRP = SUB // 16
NCHUNK = TILE_EDGES // EC
ZROWS = 392
ZITER = (ACC_ROWS // TILES) // ZROWS
WROWS = 80
WCHUNKS = N_NODES // WROWS
WITER = (WCHUNKS + TILES - 1) // TILES
BPT = BATCH // (NUM_SC * TILES)

_mesh = plsc.VectorSubcoreMesh(core_axis_name="c", subcore_axis_name="s")


@functools.partial(
    pl.kernel,
    out_type=jax.ShapeDtypeStruct((NUM_SC * N_NODES, HDIM), jnp.float32),
    mesh=_mesh,
    scratch_types=[
        pltpu.VMEM((4, EC), jnp.int32),
        pltpu.VMEM((4, EC), jnp.int32),
        pltpu.VMEM((4, EC), jnp.float32),
        pltpu.VMEM((3, NSUB, SUB), jnp.int32),
        pltpu.VMEM((3, NSUB, SUB), jnp.int32),
        pltpu.VMEM((3, EC, HDIM), jnp.float32),
        pltpu.VMEM_SHARED((ACC_ROWS, HDIM), jnp.float32),
        pltpu.SemaphoreType.DMA,
        pltpu.SemaphoreType.DMA,
        pltpu.SemaphoreType.DMA,
    ],
    compiler_params=pltpu.CompilerParams(use_tc_tiling_on_sc=False),
)
def _spmm(emb, rowa, cola, wa, out, rowb, colb, wb, colb2, idxb, rows, acc,
          sem_lin, sem_g, sem_s):
    c = lax.axis_index("c")
    s = lax.axis_index("s")
    coff = c * N_NODES

    def _zrow(i, carry):
        rows[0, i, pl.ds(0, HDIM)] = jnp.zeros((16,), jnp.float32)
        return carry
    lax.fori_loop(0, ZROWS, _zrow, 0)

    zdescs = [
        pltpu.async_copy(
            rows.at[0, pl.ds(0, ZROWS)],
            acc.at[pl.ds(s * (ACC_ROWS // TILES) + i * ZROWS, ZROWS)], sem_g)
        for i in range(ZITER)
    ]
    for zd in zdescs:
        zd.wait()
    plsc.subcore_barrier()

    base0 = s * TILE_EDGES

    def _issue_lin(cj):
        b = base0 + cj * EC
        slot = lax.rem(cj, 4) if not isinstance(cj, int) else cj % 4
        pltpu.async_copy(rowa.at[pl.ds(b, EC)], rowb.at[slot], sem_lin)
        pltpu.async_copy(cola.at[pl.ds(b, EC)], colb.at[slot], sem_lin)
        pltpu.async_copy(wa.at[pl.ds(b, EC)], wb.at[slot], sem_lin)

    def _wait_lin():
        pltpu.make_async_copy(rowa.at[pl.ds(0, EC)], rowb.at[0], sem_lin).wait()
        pltpu.make_async_copy(cola.at[pl.ds(0, EC)], colb.at[0], sem_lin).wait()
        pltpu.make_async_copy(wa.at[pl.ds(0, EC)], wb.at[0], sem_lin).wait()

    def _fixup(cj):
        s4 = lax.rem(cj, 4) if not isinstance(cj, int) else cj % 4
        s3 = lax.rem(cj, 3) if not isinstance(cj, int) else cj % 3

        def _f(k, carry):
            def _fg(g, c2):
                off = k * SUB + g * 16
                colb2[s3, k, pl.ds(g * 16, 16)] = (colb[s4, pl.ds(off, 16)]
                                                   + coff)
                idxb[s3, k, pl.ds(g * 16, 16)] = rowb[s4, pl.ds(off, 16)]
                return c2
            lax.fori_loop(0, NGRP, _fg, 0)
            return carry
        lax.fori_loop(0, NSUB, _f, 0)

    def _issue_gathers(cj):
        s3 = lax.rem(cj, 3) if not isinstance(cj, int) else cj % 3
        for k in range(NSUB):
            pltpu.async_copy(emb.at[colb2.at[s3, k]],
                             rows.at[s3, pl.ds(k * SUB, SUB)], sem_g)

    def _wait_gathers():
        pltpu.make_async_copy(emb.at[pl.ds(0, EC)], rows.at[0], sem_g).wait()

    def _drain_scatters():
        pltpu.make_async_copy(rows.at[0], acc.at[pl.ds(0, EC)], sem_s).wait()

    _issue_lin(0)
    _issue_lin(1)
    _issue_lin(2)
    _wait_lin()
    _fixup(0)
    _issue_gathers(0)
    _wait_lin()
    _fixup(1)
    _issue_gathers(1)

    def _chunk(ci, carry):
        s4 = lax.rem(ci, 4)
        s3 = lax.rem(ci, 3)

        _wait_gathers()

        def _sub(k, carry2):
            def _grp(g, carry3):
                off = k * SUB + g * 16
                w = wb[s4, pl.ds(off, 16)]
                for j in range(16):
                    e = off + j
                    wj = w.at[jnp.full((16,), j, jnp.int32)].get(
                        mode="promise_in_bounds")
                    rows[s3, e, pl.ds(0, HDIM)] = (rows[s3, e, pl.ds(0, HDIM)]
                                                   * wj)
                return carry3
            lax.fori_loop(0, NGRP, _grp, 0)
            pltpu.async_copy(rows.at[s3, pl.ds(k * SUB, SUB)],
                             acc.at[idxb.at[s3, k]], sem_s, add=True)
            return carry2
        lax.fori_loop(0, NSUB, _sub, 0)

        @pl.when(ci >= 1)
        def _():
            _drain_scatters()

        @pl.when(ci + 2 < NCHUNK)
        def _():
            _wait_lin()
            _fixup(ci + 2)
            _issue_gathers(ci + 2)

            @pl.when(ci + 3 < NCHUNK)
            def _():
                _issue_lin(ci + 3)
        return carry

    lax.fori_loop(0, NCHUNK, _chunk, 0)
    _drain_scatters()
    plsc.subcore_barrier()

    def _wb(i, carry):
        j = s + i * TILES

        @pl.when(j < WCHUNKS)
        def _():
            src = j * WROWS
            pltpu.sync_copy(acc.at[pl.ds(src, WROWS)],
                            rows.at[0, pl.ds(0, WROWS)])
            pltpu.sync_copy(rows.at[0, pl.ds(0, WROWS)],
                            out.at[pl.ds(coff + src, WROWS)])
        return carry
    lax.fori_loop(0, WITER, _wb, 0)


@functools.partial(
    pl.kernel,
    out_type=(jax.ShapeDtypeStruct((BATCH, EMBED_DIM), jnp.float32),) * 3,
    mesh=_mesh,
    scratch_types=[
        pltpu.VMEM((BPT,), jnp.int32),
        pltpu.VMEM((BPT,), jnp.int32),
        pltpu.VMEM((8, BPT, HDIM), jnp.float32),
        pltpu.VMEM((BPT, EMBED_DIM), jnp.float32),
        pltpu.SemaphoreType.DMA,
    ],
    compiler_params=pltpu.CompilerParams(use_tc_tiling_on_sc=False),
)
def _final(t0, t1, t2, t3, usr, pos, neg, ou, op, on, idlo, idhi, bufs, obuf,
           sem):
    c = lax.axis_index("c")
    s = lax.axis_index("s")
    base = (s * NUM_SC + c) * BPT
    for ids, off, outref in ((usr, 0, ou), (pos, N_USERS, op),
                             (neg, N_USERS, on)):
        pltpu.sync_copy(ids.at[pl.ds(base, BPT)], idlo)

        def _adj(g, carry):
            v = idlo[pl.ds(g * 16, 16)] + off
            idlo[pl.ds(g * 16, 16)] = v
            idhi[pl.ds(g * 16, 16)] = v + N_NODES
            return carry
        lax.fori_loop(0, BPT // 16, _adj, 0)
        descs = []
        for ti, t in enumerate((t0, t1, t2, t3)):
            descs.append(pltpu.async_copy(t.at[idlo], bufs.at[2 * ti], sem))
            descs.append(pltpu.async_copy(t.at[idhi], bufs.at[2 * ti + 1],
                                          sem))
        for d in descs:
            d.wait()

        def _mean(r, carry):
            lo = (bufs[0, r, pl.ds(0, HDIM)] + bufs[2, r, pl.ds(0, HDIM)]
                  + bufs[4, r, pl.ds(0, HDIM)]
                  + bufs[6, r, pl.ds(0, HDIM)]) * 0.25
            hi = (bufs[1, r, pl.ds(0, HDIM)] + bufs[3, r, pl.ds(0, HDIM)]
                  + bufs[5, r, pl.ds(0, HDIM)]
                  + bufs[7, r, pl.ds(0, HDIM)]) * 0.25
            obuf[r, pl.ds(0, HDIM)] = lo
            obuf[r, pl.ds(HDIM, HDIM)] = hi
            return carry
        lax.fori_loop(0, BPT, _mean, 0)
        pltpu.sync_copy(obuf, outref.at[pl.ds(base, BPT)])


def kernel(users, pos_items, neg_items, edge_index, edge_weight, user_emb,
           item_emb):
    row = edge_index[0]
    col = edge_index[1]
    all_emb = jnp.concatenate([user_emb, item_emb], axis=0)
    e0 = jnp.concatenate([all_emb[:, :HDIM], all_emb[:, HDIM:]], axis=0)
    e1 = _spmm(e0, row, col, edge_weight)
    e2 = _spmm(e1, row, col, edge_weight)
    e3 = _spmm(e2, row, col, edge_weight)
    return _final(e0, e1, e2, e3, users, pos_items, neg_items)


# speedup vs baseline: 2.8863x; 1.0294x over previous
import functools

import jax
import jax.numpy as jnp
from jax import lax
from jax.experimental import pallas as pl
from jax.experimental.pallas import tpu as pltpu
from jax.experimental.pallas import tpu_sc as plsc

N_USERS = 25000
N_ITEMS = 75000
N_NODES = N_USERS + N_ITEMS
EMBED_DIM = 32
HDIM = EMBED_DIM // 2
N_EDGES = 1_600_000
BATCH = 4096

NUM_SC = 2
TILES = 16
ACC_ROWS = 100352
TILE_EDGES = N_EDGES // TILES
EC = 400
SUB = 80
NSUB = EC // SUB
NGRP = SUB // 16
NCHUNK = TILE_EDGES // EC
ZROWS = 392
ZITER = (ACC_ROWS // TILES) // ZROWS
WROWS = 80
WCHUNKS = N_NODES // WROWS
WITER = (WCHUNKS + TILES - 1) // TILES
BPT = BATCH // (NUM_SC * TILES)

_mesh = plsc.VectorSubcoreMesh(core_axis_name="c", subcore_axis_name="s")


@functools.partial(
    pl.kernel,
    out_type=(jax.ShapeDtypeStruct((NUM_SC * N_NODES, HDIM), jnp.float32),) * 3,
    mesh=_mesh,
    scratch_types=[
        pltpu.VMEM((4, EC), jnp.int32),
        pltpu.VMEM((4, EC), jnp.int32),
        pltpu.VMEM((4, EC), jnp.float32),
        pltpu.VMEM((3, NSUB, SUB), jnp.int32),
        pltpu.VMEM((3, NSUB, SUB), jnp.int32),
        pltpu.VMEM((3, EC, HDIM), jnp.float32),
        pltpu.VMEM_SHARED((ACC_ROWS, HDIM), jnp.float32),
        pltpu.SemaphoreType.DMA,
        pltpu.SemaphoreType.DMA,
        pltpu.SemaphoreType.DMA,
    ],
    compiler_params=pltpu.CompilerParams(use_tc_tiling_on_sc=False),
)
def _layers(e0v, rowa, cola, wa, e1, e2, e3, rowb, colb, wb, colb2, idxb,
            rows, acc, sem_lin, sem_g, sem_s):
    c = lax.axis_index("c")
    s = lax.axis_index("s")
    coff = c * N_NODES
    base0 = s * TILE_EDGES

    def _issue_lin(cj):
        b = base0 + cj * EC
        slot = lax.rem(cj, 4) if not isinstance(cj, int) else cj % 4
        pltpu.async_copy(rowa.at[pl.ds(b, EC)], rowb.at[slot], sem_lin)
        pltpu.async_copy(cola.at[pl.ds(b, EC)], colb.at[slot], sem_lin)
        pltpu.async_copy(wa.at[pl.ds(b, EC)], wb.at[slot], sem_lin)

    def _wait_lin():
        pltpu.make_async_copy(rowa.at[pl.ds(0, EC)], rowb.at[0], sem_lin).wait()
        pltpu.make_async_copy(cola.at[pl.ds(0, EC)], colb.at[0], sem_lin).wait()
        pltpu.make_async_copy(wa.at[pl.ds(0, EC)], wb.at[0], sem_lin).wait()

    def _one_layer(emb, out, interleaved):
        def _zrow(i, carry):
            rows[0, i, pl.ds(0, HDIM)] = jnp.zeros((16,), jnp.float32)
            return carry
        lax.fori_loop(0, ZROWS, _zrow, 0)

        zdescs = [
            pltpu.async_copy(
                rows.at[0, pl.ds(0, ZROWS)],
                acc.at[pl.ds(s * (ACC_ROWS // TILES) + i * ZROWS, ZROWS)],
                sem_g)
            for i in range(ZITER)
        ]
        for zd in zdescs:
            zd.wait()
        plsc.subcore_barrier()

        def _fixup(cj):
            s4 = lax.rem(cj, 4) if not isinstance(cj, int) else cj % 4
            s3 = lax.rem(cj, 3) if not isinstance(cj, int) else cj % 3

            def _f(k, carry):
                def _fg(g, c2):
                    off = k * SUB + g * 16
                    cv = colb[s4, pl.ds(off, 16)]
                    gidx = 2 * cv + c if interleaved else cv + coff
                    colb2[s3, k, pl.ds(g * 16, 16)] = gidx
                    idxb[s3, k, pl.ds(g * 16, 16)] = rowb[s4, pl.ds(off, 16)]
                    return c2
                lax.fori_loop(0, NGRP, _fg, 0)
                return carry
            lax.fori_loop(0, NSUB, _f, 0)

        def _issue_gathers(cj):
            s3 = lax.rem(cj, 3) if not isinstance(cj, int) else cj % 3
            for k in range(NSUB):
                pltpu.async_copy(emb.at[colb2.at[s3, k]],
                                 rows.at[s3, pl.ds(k * SUB, SUB)], sem_g)

        def _wait_gathers():
            pltpu.make_async_copy(emb.at[pl.ds(0, EC)], rows.at[0],
                                  sem_g).wait()

        def _drain_scatters():
            pltpu.make_async_copy(rows.at[0], acc.at[pl.ds(0, EC)],
                                  sem_s).wait()

        _issue_lin(0)
        _issue_lin(1)
        _issue_lin(2)
        _wait_lin()
        _fixup(0)
        _issue_gathers(0)
        _wait_lin()
        _fixup(1)
        _issue_gathers(1)

        def _chunk(ci, carry):
            s4 = lax.rem(ci, 4)
            s3 = lax.rem(ci, 3)

            _wait_gathers()

            def _sub(k, carry2):
                def _grp(g, carry3):
                    off = k * SUB + g * 16
                    w = wb[s4, pl.ds(off, 16)]
                    for j in range(16):
                        e = off + j
                        wj = w.at[jnp.full((16,), j, jnp.int32)].get(
                            mode="promise_in_bounds")
                        rows[s3, e, pl.ds(0, HDIM)] = (
                            rows[s3, e, pl.ds(0, HDIM)] * wj)
                    return carry3
                lax.fori_loop(0, NGRP, _grp, 0)
                pltpu.async_copy(rows.at[s3, pl.ds(k * SUB, SUB)],
                                 acc.at[idxb.at[s3, k]], sem_s, add=True)
                return carry2
            lax.fori_loop(0, NSUB, _sub, 0)

            @pl.when(ci >= 1)
            def _():
                _drain_scatters()

            @pl.when(ci + 2 < NCHUNK)
            def _():
                _wait_lin()
                _fixup(ci + 2)
                _issue_gathers(ci + 2)

                @pl.when(ci + 3 < NCHUNK)
                def _():
                    _issue_lin(ci + 3)
            return carry

        lax.fori_loop(0, NCHUNK, _chunk, 0)
        _drain_scatters()
        plsc.subcore_barrier()

        def _wb(i, carry):
            j = s + i * TILES

            @pl.when(j < WCHUNKS)
            def _():
                src = j * WROWS
                pltpu.sync_copy(acc.at[pl.ds(src, WROWS)],
                                rows.at[0, pl.ds(0, WROWS)])
                pltpu.sync_copy(rows.at[0, pl.ds(0, WROWS)],
                                out.at[pl.ds(coff + src, WROWS)])
            return carry
        lax.fori_loop(0, WITER, _wb, 0)
        plsc.subcore_barrier()

    _one_layer(e0v, e1, True)
    _one_layer(e1, e2, False)
    _one_layer(e2, e3, False)


@functools.partial(
    pl.kernel,
    out_type=(jax.ShapeDtypeStruct((BATCH, EMBED_DIM), jnp.float32),) * 3,
    mesh=_mesh,
    scratch_types=[
        pltpu.VMEM((BPT,), jnp.int32),
        pltpu.VMEM((BPT,), jnp.int32),
        pltpu.VMEM((BPT,), jnp.int32),
        pltpu.VMEM((BPT,), jnp.int32),
        pltpu.VMEM((8, BPT, HDIM), jnp.float32),
        pltpu.VMEM((BPT, EMBED_DIM), jnp.float32),
        pltpu.SemaphoreType.DMA,
    ],
    compiler_params=pltpu.CompilerParams(use_tc_tiling_on_sc=False),
)
def _final(t0, t1, t2, t3, usr, pos, neg, ou, op, on, idlo, idhi, idlo0,
           idhi0, bufs, obuf, sem):
    c = lax.axis_index("c")
    s = lax.axis_index("s")
    base = (s * NUM_SC + c) * BPT
    for ids, off, outref in ((usr, 0, ou), (pos, N_USERS, op),
                             (neg, N_USERS, on)):
        pltpu.sync_copy(ids.at[pl.ds(base, BPT)], idlo)

        def _adj(g, carry):
            v = idlo[pl.ds(g * 16, 16)] + off
            idlo[pl.ds(g * 16, 16)] = v
            idhi[pl.ds(g * 16, 16)] = v + N_NODES
            idlo0[pl.ds(g * 16, 16)] = 2 * v
            idhi0[pl.ds(g * 16, 16)] = 2 * v + 1
            return carry
        lax.fori_loop(0, BPT // 16, _adj, 0)
        descs = [pltpu.async_copy(t0.at[idlo0], bufs.at[0], sem),
                 pltpu.async_copy(t0.at[idhi0], bufs.at[1], sem)]
        for ti, t in enumerate((t1, t2, t3)):
            descs.append(pltpu.async_copy(t.at[idlo], bufs.at[2 * ti + 2],
                                          sem))
            descs.append(pltpu.async_copy(t.at[idhi], bufs.at[2 * ti + 3],
                                          sem))
        for d in descs:
            d.wait()

        def _mean(r, carry):
            lo = (bufs[0, r, pl.ds(0, HDIM)] + bufs[2, r, pl.ds(0, HDIM)]
                  + bufs[4, r, pl.ds(0, HDIM)]
                  + bufs[6, r, pl.ds(0, HDIM)]) * 0.25
            hi = (bufs[1, r, pl.ds(0, HDIM)] + bufs[3, r, pl.ds(0, HDIM)]
                  + bufs[5, r, pl.ds(0, HDIM)]
                  + bufs[7, r, pl.ds(0, HDIM)]) * 0.25
            obuf[r, pl.ds(0, HDIM)] = lo
            obuf[r, pl.ds(HDIM, HDIM)] = hi
            return carry
        lax.fori_loop(0, BPT, _mean, 0)
        pltpu.sync_copy(obuf, outref.at[pl.ds(base, BPT)])


def kernel(users, pos_items, neg_items, edge_index, edge_weight, user_emb,
           item_emb):
    row = edge_index[0]
    col = edge_index[1]
    all_emb = jnp.concatenate([user_emb, item_emb], axis=0)
    e0v = all_emb.reshape(NUM_SC * N_NODES, HDIM)
    e1, e2, e3 = _layers(e0v, row, col, edge_weight)
    return _final(e0v, e1, e2, e3, users, pos_items, neg_items)


# speedup vs baseline: 2.9014x; 1.0052x over previous
import functools

import jax
import jax.numpy as jnp
from jax import lax
from jax.experimental import pallas as pl
from jax.experimental.pallas import tpu as pltpu
from jax.experimental.pallas import tpu_sc as plsc

N_USERS = 25000
N_ITEMS = 75000
N_NODES = N_USERS + N_ITEMS
EMBED_DIM = 32
HDIM = EMBED_DIM // 2
N_EDGES = 1_600_000
BATCH = 4096

NUM_SC = 2
TILES = 16
ACC_ROWS = 100352
TILE_EDGES = N_EDGES // TILES
EC = 400
SUB = 80
NSUB = EC // SUB
NGRP = SUB // 16
NCHUNK = TILE_EDGES // EC
ZROWS = 392
ZITER = (ACC_ROWS // TILES) // ZROWS
WROWS = 80
WCHUNKS = N_NODES // WROWS
WITER = (WCHUNKS + TILES - 1) // TILES
BPT = BATCH // (NUM_SC * TILES)

_mesh = plsc.VectorSubcoreMesh(core_axis_name="c", subcore_axis_name="s")


@functools.partial(
    pl.kernel,
    out_type=(jax.ShapeDtypeStruct((NUM_SC * N_NODES, HDIM), jnp.float32),) * 3,
    mesh=_mesh,
    scratch_types=[
        pltpu.VMEM((4, EC), jnp.int32),
        pltpu.VMEM((4, EC), jnp.int32),
        pltpu.VMEM((4, EC), jnp.float32),
        pltpu.VMEM((3, NSUB, SUB), jnp.int32),
        pltpu.VMEM((3, NSUB, SUB), jnp.int32),
        pltpu.VMEM((3, EC, HDIM), jnp.float32),
        pltpu.VMEM_SHARED((ACC_ROWS, HDIM), jnp.float32),
        pltpu.SemaphoreType.DMA,
        pltpu.SemaphoreType.DMA,
        pltpu.SemaphoreType.DMA,
        pltpu.SemaphoreType.DMA,
    ],
    compiler_params=pltpu.CompilerParams(use_tc_tiling_on_sc=False),
)
def _layers(e0v, rowa, cola, wa, e1, e2, e3, rowb, colb, wb, colb2, idxb,
            rows, acc, sem_lin, sem_g, sem_s, sem_z):
    c = lax.axis_index("c")
    s = lax.axis_index("s")
    coff = c * N_NODES
    base0 = s * TILE_EDGES

    def _issue_lin(cj):
        b = base0 + cj * EC
        slot = lax.rem(cj, 4) if not isinstance(cj, int) else cj % 4
        pltpu.async_copy(rowa.at[pl.ds(b, EC)], rowb.at[slot], sem_lin)
        pltpu.async_copy(cola.at[pl.ds(b, EC)], colb.at[slot], sem_lin)
        pltpu.async_copy(wa.at[pl.ds(b, EC)], wb.at[slot], sem_lin)

    def _wait_lin():
        pltpu.make_async_copy(rowa.at[pl.ds(0, EC)], rowb.at[0], sem_lin).wait()
        pltpu.make_async_copy(cola.at[pl.ds(0, EC)], colb.at[0], sem_lin).wait()
        pltpu.make_async_copy(wa.at[pl.ds(0, EC)], wb.at[0], sem_lin).wait()

    def _one_layer(emb, out, interleaved):
        def _zrow(i, carry):
            rows[2, i, pl.ds(0, HDIM)] = jnp.zeros((16,), jnp.float32)
            return carry
        lax.fori_loop(0, ZROWS, _zrow, 0)

        zdescs = [
            pltpu.async_copy(
                rows.at[2, pl.ds(0, ZROWS)],
                acc.at[pl.ds(s * (ACC_ROWS // TILES) + i * ZROWS, ZROWS)],
                sem_z)
            for i in range(ZITER)
        ]

        def _fixup(cj):
            s4 = lax.rem(cj, 4) if not isinstance(cj, int) else cj % 4
            s3 = lax.rem(cj, 3) if not isinstance(cj, int) else cj % 3

            def _f(k, carry):
                def _fg(g, c2):
                    off = k * SUB + g * 16
                    cv = colb[s4, pl.ds(off, 16)]
                    gidx = 2 * cv + c if interleaved else cv + coff
                    colb2[s3, k, pl.ds(g * 16, 16)] = gidx
                    idxb[s3, k, pl.ds(g * 16, 16)] = rowb[s4, pl.ds(off, 16)]
                    return c2
                lax.fori_loop(0, NGRP, _fg, 0)
                return carry
            lax.fori_loop(0, NSUB, _f, 0)

        def _issue_gathers(cj):
            s3 = lax.rem(cj, 3) if not isinstance(cj, int) else cj % 3
            for k in range(NSUB):
                pltpu.async_copy(emb.at[colb2.at[s3, k]],
                                 rows.at[s3, pl.ds(k * SUB, SUB)], sem_g)

        def _wait_gathers():
            pltpu.make_async_copy(emb.at[pl.ds(0, EC)], rows.at[0],
                                  sem_g).wait()

        def _drain_scatters():
            pltpu.make_async_copy(rows.at[0], acc.at[pl.ds(0, EC)],
                                  sem_s).wait()

        _issue_lin(0)
        _issue_lin(1)
        _issue_lin(2)
        _wait_lin()
        _fixup(0)
        _issue_gathers(0)
        _wait_lin()
        _fixup(1)
        _issue_gathers(1)
        for zd in zdescs:
            zd.wait()
        plsc.subcore_barrier()

        def _chunk(ci, carry):
            s4 = lax.rem(ci, 4)
            s3 = lax.rem(ci, 3)

            _wait_gathers()

            def _sub(k, carry2):
                def _grp(g, carry3):
                    off = k * SUB + g * 16
                    w = wb[s4, pl.ds(off, 16)]
                    for j in range(16):
                        e = off + j
                        wj = w.at[jnp.full((16,), j, jnp.int32)].get(
                            mode="promise_in_bounds")
                        rows[s3, e, pl.ds(0, HDIM)] = (
                            rows[s3, e, pl.ds(0, HDIM)] * wj)
                    return carry3
                lax.fori_loop(0, NGRP, _grp, 0)
                pltpu.async_copy(rows.at[s3, pl.ds(k * SUB, SUB)],
                                 acc.at[idxb.at[s3, k]], sem_s, add=True)
                return carry2
            lax.fori_loop(0, NSUB, _sub, 0)

            @pl.when(ci >= 1)
            def _():
                _drain_scatters()

            @pl.when(ci + 2 < NCHUNK)
            def _():
                _wait_lin()
                _fixup(ci + 2)
                _issue_gathers(ci + 2)

                @pl.when(ci + 3 < NCHUNK)
                def _():
                    _issue_lin(ci + 3)
            return carry

        lax.fori_loop(0, NCHUNK, _chunk, 0)
        _drain_scatters()
        plsc.subcore_barrier()

        def _wb(i, carry):
            j = s + i * TILES

            @pl.when(j < WCHUNKS)
            def _():
                src = j * WROWS
                pltpu.sync_copy(acc.at[pl.ds(src, WROWS)],
                                rows.at[0, pl.ds(0, WROWS)])
                pltpu.sync_copy(rows.at[0, pl.ds(0, WROWS)],
                                out.at[pl.ds(coff + src, WROWS)])
            return carry
        lax.fori_loop(0, WITER, _wb, 0)
        plsc.subcore_barrier()

    _one_layer(e0v, e1, True)
    _one_layer(e1, e2, False)
    _one_layer(e2, e3, False)


@functools.partial(
    pl.kernel,
    out_type=(jax.ShapeDtypeStruct((BATCH, EMBED_DIM), jnp.float32),) * 3,
    mesh=_mesh,
    scratch_types=[
        pltpu.VMEM((BPT,), jnp.int32),
        pltpu.VMEM((BPT,), jnp.int32),
        pltpu.VMEM((BPT,), jnp.int32),
        pltpu.VMEM((BPT,), jnp.int32),
        pltpu.VMEM((8, BPT, HDIM), jnp.float32),
        pltpu.VMEM((BPT, EMBED_DIM), jnp.float32),
        pltpu.SemaphoreType.DMA,
    ],
    compiler_params=pltpu.CompilerParams(use_tc_tiling_on_sc=False),
)
def _final(t0, t1, t2, t3, usr, pos, neg, ou, op, on, idlo, idhi, idlo0,
           idhi0, bufs, obuf, sem):
    c = lax.axis_index("c")
    s = lax.axis_index("s")
    base = (s * NUM_SC + c) * BPT
    for ids, off, outref in ((usr, 0, ou), (pos, N_USERS, op),
                             (neg, N_USERS, on)):
        pltpu.sync_copy(ids.at[pl.ds(base, BPT)], idlo)

        def _adj(g, carry):
            v = idlo[pl.ds(g * 16, 16)] + off
            idlo[pl.ds(g * 16, 16)] = v
            idhi[pl.ds(g * 16, 16)] = v + N_NODES
            idlo0[pl.ds(g * 16, 16)] = 2 * v
            idhi0[pl.ds(g * 16, 16)] = 2 * v + 1
            return carry
        lax.fori_loop(0, BPT // 16, _adj, 0)
        descs = [pltpu.async_copy(t0.at[idlo0], bufs.at[0], sem),
                 pltpu.async_copy(t0.at[idhi0], bufs.at[1], sem)]
        for ti, t in enumerate((t1, t2, t3)):
            descs.append(pltpu.async_copy(t.at[idlo], bufs.at[2 * ti + 2],
                                          sem))
            descs.append(pltpu.async_copy(t.at[idhi], bufs.at[2 * ti + 3],
                                          sem))
        for d in descs:
            d.wait()

        def _mean(r, carry):
            lo = (bufs[0, r, pl.ds(0, HDIM)] + bufs[2, r, pl.ds(0, HDIM)]
                  + bufs[4, r, pl.ds(0, HDIM)]
                  + bufs[6, r, pl.ds(0, HDIM)]) * 0.25
            hi = (bufs[1, r, pl.ds(0, HDIM)] + bufs[3, r, pl.ds(0, HDIM)]
                  + bufs[5, r, pl.ds(0, HDIM)]
                  + bufs[7, r, pl.ds(0, HDIM)]) * 0.25
            obuf[r, pl.ds(0, HDIM)] = lo
            obuf[r, pl.ds(HDIM, HDIM)] = hi
            return carry
        lax.fori_loop(0, BPT, _mean, 0)
        pltpu.sync_copy(obuf, outref.at[pl.ds(base, BPT)])


def kernel(users, pos_items, neg_items, edge_index, edge_weight, user_emb,
           item_emb):
    row = edge_index[0]
    col = edge_index[1]
    all_emb = jnp.concatenate([user_emb, item_emb], axis=0)
    e0v = all_emb.reshape(NUM_SC * N_NODES, HDIM)
    e1, e2, e3 = _layers(e0v, row, col, edge_weight)
    return _final(e0v, e1, e2, e3, users, pos_items, neg_items)
